# baseline (speedup 1.0000x reference)
import functools

import jax
import jax.numpy as jnp
from jax import lax
from jax.experimental import pallas as pl
from jax.experimental.pallas import tpu as pltpu
from jax.experimental.pallas import tpu_sc as plsc

N = 65536
T = 2097152

NC = 2
NS = 16
L = 16
NW = NC * NS

PER_W = T // NW
CHUNK = 2048
N_CHUNKS = PER_W // CHUNK
GROUPS = CHUNK // L
DMA_SLICE = 128
N_DMA = CHUNK // DMA_SLICE

_INV_PI = 0.3183098861837907
_PI_HI = 3.140625
_PI_LO = 9.676535897932795e-4
_S1, _S2, _S3, _S4 = -1.6666667e-1, 8.3333333e-3, -1.9841270e-4, 2.7557319e-6
_C1, _C2, _C3, _C4, _C5 = -0.5, 4.1666667e-2, -1.3888889e-3, 2.4801587e-5, -2.7557319e-7


def _sincos(x):
    y = x * _INV_PI + 0.5
    ki = y.astype(jnp.int32)
    kf = ki.astype(jnp.float32)
    adj = kf > y
    kf = kf - jnp.where(adj, 1.0, 0.0)
    ki = ki - jnp.where(adj, 1, 0)
    r = x - kf * _PI_HI
    r = r - kf * _PI_LO
    r2 = r * r
    ps = (((_S4 * r2 + _S3) * r2 + _S2) * r2 + _S1) * r2 + 1.0
    pc = ((((_C5 * r2 + _C4) * r2 + _C3) * r2 + _C2) * r2 + _C1) * r2 + 1.0
    sinr = r * ps
    sgn = jnp.where((ki & 1) != 0, -1.0, 1.0)
    return sinr, pc, sgn


def _body(table_hbm, t_hbm, gx_hbm, gy_hbm, gz_hbm, re_hbm, im_hbm,
          t_v, idx_v, frac_v, rows_v, gx_v, gy_v, gz_v, re_v, im_v, sem):
    wid = lax.axis_index("s") * NC + lax.axis_index("c")
    w_base = wid * PER_W

    def chunk_body(c, carry):
        base = w_base + c * CHUNK
        pltpu.sync_copy(t_hbm.at[pl.ds(base, CHUNK)], t_v)

        @plsc.parallel_loop(0, CHUNK, L, unroll=8)
        def pass1(i):
            tv = t_v[pl.ds(i, L)]
            ti = tv.astype(jnp.int32)
            ti = jnp.minimum(jnp.maximum(ti, 0), N - 2)
            idx_v[pl.ds(i, L)] = ti
            frac_v[pl.ds(i, L)] = tv - ti.astype(jnp.float32)

        copies = []
        for d in range(N_DMA):
            copies.append(pltpu.async_copy(
                table_hbm.at[idx_v.at[pl.ds(d * DMA_SLICE, DMA_SLICE)]],
                rows_v.at[pl.ds(d * DMA_SLICE, DMA_SLICE)],
                sem))
        for cp in copies:
            cp.wait()

        @plsc.parallel_loop(0, CHUNK, L, unroll=8)
        def pass2(i):
            sl = pl.ds(i, L)
            fr = frac_v[sl]
            ri = lax.iota(jnp.int32, L) + i

            def col(cc):
                ci = jnp.full((L,), cc, jnp.int32)
                return plsc.load_gather(rows_v, [ri, ci])

            def lerp(cc):
                a = col(cc)
                b = col(cc + 5)
                return a + fr * (b - a)

            gx_v[sl] = lerp(0)
            gy_v[sl] = lerp(1)
            gz_v[sl] = lerp(2)
            amp = lerp(3)
            ph = lerp(4)
            sinr, cosr, sgn = _sincos(ph)
            amps = amp * sgn
            re_v[sl] = amps * cosr
            im_v[sl] = amps * sinr

        pltpu.sync_copy(gx_v, gx_hbm.at[pl.ds(base, CHUNK)])
        pltpu.sync_copy(gy_v, gy_hbm.at[pl.ds(base, CHUNK)])
        pltpu.sync_copy(gz_v, gz_hbm.at[pl.ds(base, CHUNK)])
        pltpu.sync_copy(re_v, re_hbm.at[pl.ds(base, CHUNK)])
        pltpu.sync_copy(im_v, im_hbm.at[pl.ds(base, CHUNK)])
        return carry

    lax.fori_loop(0, N_CHUNKS, chunk_body, 0)


_mesh = plsc.VectorSubcoreMesh(
    core_axis_name="c", subcore_axis_name="s", num_cores=NC, num_subcores=NS)

_sc_interp = functools.partial(
    pl.kernel,
    out_type=[jax.ShapeDtypeStruct((T,), jnp.float32)] * 5,
    mesh=_mesh,
    compiler_params=pltpu.CompilerParams(
        use_tc_tiling_on_sc=False, needs_layout_passes=False),
    scratch_types=[
        pltpu.VMEM((CHUNK,), jnp.float32),
        pltpu.VMEM((CHUNK,), jnp.int32),
        pltpu.VMEM((CHUNK,), jnp.float32),
        pltpu.VMEM((CHUNK, 16), jnp.float32),
        pltpu.VMEM((CHUNK,), jnp.float32),
        pltpu.VMEM((CHUNK,), jnp.float32),
        pltpu.VMEM((CHUNK,), jnp.float32),
        pltpu.VMEM((CHUNK,), jnp.float32),
        pltpu.VMEM((CHUNK,), jnp.float32),
        pltpu.SemaphoreType.DMA,
    ],
)(_body)


_BT = 65536


def _stack_body(gx_ref, gy_ref, gz_ref, out_ref):
    out_ref[0, :] = gx_ref[:]
    out_ref[1, :] = gy_ref[:]
    out_ref[2, :] = gz_ref[:]


_stack3 = pl.pallas_call(
    _stack_body,
    grid=(T // _BT,),
    in_specs=[pl.BlockSpec((_BT,), lambda j: (j,))] * 3,
    out_specs=pl.BlockSpec((3, _BT), lambda j: (0, j)),
    out_shape=jax.ShapeDtypeStruct((3, T), jnp.float32),
)


def kernel(time_points, gradients, rf_amplitude, rf_phase, adc_mask, t):
    rows5 = jnp.concatenate(
        [gradients, rf_amplitude[:, None], rf_phase[:, None]], axis=1)
    nxt = jnp.concatenate([rows5[1:], rows5[-1:]], axis=0)
    packed = jnp.concatenate(
        [rows5, nxt, jnp.zeros((N, 6), jnp.float32)], axis=1)
    gx, gy, gz, re, im = _sc_interp(packed, t)
    g = _stack3(gx, gy, gz)
    rf = lax.complex(re, im)
    return (g, rf)


# speedup vs baseline: 1.0504x; 1.0504x over previous
import functools

import jax
import jax.numpy as jnp
from jax import lax
from jax.experimental import pallas as pl
from jax.experimental.pallas import tpu as pltpu
from jax.experimental.pallas import tpu_sc as plsc

N = 65536
T = 2097152

NC = 2
NS = 16
L = 16
NW = NC * NS

PER_W = T // NW
CHUNK = 2048
N_CHUNKS = PER_W // CHUNK
GROUPS = CHUNK // L
DMA_SLICE = 128
N_DMA = CHUNK // DMA_SLICE

_INV_PI = 0.3183098861837907
_PI_HI = 3.140625
_PI_LO = 9.676535897932795e-4
_S1, _S2, _S3, _S4 = -1.6666667e-1, 8.3333333e-3, -1.9841270e-4, 2.7557319e-6
_C1, _C2, _C3, _C4, _C5 = -0.5, 4.1666667e-2, -1.3888889e-3, 2.4801587e-5, -2.7557319e-7


def _sincos(x):
    y = x * _INV_PI + 0.5
    ki = y.astype(jnp.int32)
    kf = ki.astype(jnp.float32)
    adj = kf > y
    kf = kf - jnp.where(adj, 1.0, 0.0)
    ki = ki - jnp.where(adj, 1, 0)
    r = x - kf * _PI_HI
    r = r - kf * _PI_LO
    r2 = r * r
    ps = (((_S4 * r2 + _S3) * r2 + _S2) * r2 + _S1) * r2 + 1.0
    pc = ((((_C5 * r2 + _C4) * r2 + _C3) * r2 + _C2) * r2 + _C1) * r2 + 1.0
    sinr = r * ps
    sgn = jnp.where((ki & 1) != 0, -1.0, 1.0)
    return sinr, pc, sgn


def _body(table_hbm, t_hbm, gx_hbm, gy_hbm, gz_hbm, re_hbm, im_hbm,
          t_v, idx_v, frac_v, rows_v, gx_v, gy_v, gz_v, re_v, im_v, sem):
    wid = lax.axis_index("s") * NC + lax.axis_index("c")
    w_base = wid * PER_W

    def chunk_body(c, carry):
        base = w_base + c * CHUNK
        pltpu.sync_copy(t_hbm.at[pl.ds(base, CHUNK)], t_v)

        @plsc.parallel_loop(0, CHUNK, L, unroll=4)
        def pass1(i):
            tv = t_v[pl.ds(i, L)]
            ti = tv.astype(jnp.int32)
            ti = jnp.minimum(jnp.maximum(ti, 0), N - 2)
            idx_v[pl.ds(i, L)] = ti
            frac_v[pl.ds(i, L)] = tv - ti.astype(jnp.float32)

        copies = []
        for d in range(N_DMA):
            copies.append(pltpu.async_copy(
                table_hbm.at[idx_v.at[pl.ds(d * DMA_SLICE, DMA_SLICE)]],
                rows_v.at[pl.ds(d * DMA_SLICE, DMA_SLICE)],
                sem))
        for cp in copies:
            cp.wait()

        @plsc.parallel_loop(0, CHUNK, L, unroll=4)
        def pass2(i):
            sl = pl.ds(i, L)
            fr = frac_v[sl]
            ri = lax.iota(jnp.int32, L) + i

            def col(cc):
                ci = jnp.full((L,), cc, jnp.int32)
                return plsc.load_gather(rows_v, [ri, ci])

            def lerp(cc):
                a = col(cc)
                b = col(cc + 5)
                return a + fr * (b - a)

            gx_v[sl] = lerp(0)
            gy_v[sl] = lerp(1)
            gz_v[sl] = lerp(2)
            amp = lerp(3)
            ph = lerp(4)
            sinr, cosr, sgn = _sincos(ph)
            amps = amp * sgn
            re_v[sl] = amps * cosr
            im_v[sl] = amps * sinr

        pltpu.sync_copy(gx_v, gx_hbm.at[pl.ds(base, CHUNK)])
        pltpu.sync_copy(gy_v, gy_hbm.at[pl.ds(base, CHUNK)])
        pltpu.sync_copy(gz_v, gz_hbm.at[pl.ds(base, CHUNK)])
        pltpu.sync_copy(re_v, re_hbm.at[pl.ds(base, CHUNK)])
        pltpu.sync_copy(im_v, im_hbm.at[pl.ds(base, CHUNK)])
        return carry

    lax.fori_loop(0, N_CHUNKS, chunk_body, 0)


_mesh = plsc.VectorSubcoreMesh(
    core_axis_name="c", subcore_axis_name="s", num_cores=NC, num_subcores=NS)

_sc_interp = functools.partial(
    pl.kernel,
    out_type=[jax.ShapeDtypeStruct((T,), jnp.float32)] * 5,
    mesh=_mesh,
    compiler_params=pltpu.CompilerParams(
        use_tc_tiling_on_sc=False, needs_layout_passes=False),
    scratch_types=[
        pltpu.VMEM((CHUNK,), jnp.float32),
        pltpu.VMEM((CHUNK,), jnp.int32),
        pltpu.VMEM((CHUNK,), jnp.float32),
        pltpu.VMEM((CHUNK, 16), jnp.float32),
        pltpu.VMEM((CHUNK,), jnp.float32),
        pltpu.VMEM((CHUNK,), jnp.float32),
        pltpu.VMEM((CHUNK,), jnp.float32),
        pltpu.VMEM((CHUNK,), jnp.float32),
        pltpu.VMEM((CHUNK,), jnp.float32),
        pltpu.SemaphoreType.DMA,
    ],
)(_body)


_BT = 65536


def _stack_body(gx_ref, gy_ref, gz_ref, out_ref):
    out_ref[0, :] = gx_ref[:]
    out_ref[1, :] = gy_ref[:]
    out_ref[2, :] = gz_ref[:]


_stack3 = pl.pallas_call(
    _stack_body,
    grid=(T // _BT,),
    in_specs=[pl.BlockSpec((_BT,), lambda j: (j,))] * 3,
    out_specs=pl.BlockSpec((3, _BT), lambda j: (0, j)),
    out_shape=jax.ShapeDtypeStruct((3, T), jnp.float32),
)


def kernel(time_points, gradients, rf_amplitude, rf_phase, adc_mask, t):
    rows5 = jnp.concatenate(
        [gradients, rf_amplitude[:, None], rf_phase[:, None]], axis=1)
    nxt = jnp.concatenate([rows5[1:], rows5[-1:]], axis=0)
    packed = jnp.concatenate(
        [rows5, nxt, jnp.zeros((N, 6), jnp.float32)], axis=1)
    gx, gy, gz, re, im = _sc_interp(packed, t)
    g = _stack3(gx, gy, gz)
    rf = lax.complex(re, im)
    return (g, rf)


# speedup vs baseline: 1.2751x; 1.2139x over previous
import functools

import jax
import jax.numpy as jnp
from jax import lax
from jax.experimental import pallas as pl
from jax.experimental.pallas import tpu as pltpu
from jax.experimental.pallas import tpu_sc as plsc

N = 65536
T = 2097152

NC = 2
NS = 16
L = 16
NW = NC * NS

PER_W = T // NW
CHUNK = 2048
N_CHUNKS = PER_W // CHUNK
GROUPS = CHUNK // L
DMA_SLICE = 128
N_DMA = CHUNK // DMA_SLICE

_INV_PI = 0.3183098861837907
_PI_HI = 3.140625
_PI_LO = 9.676535897932795e-4
_S1, _S2, _S3, _S4 = -1.6666667e-1, 8.3333333e-3, -1.9841270e-4, 2.7557319e-6
_C1, _C2, _C3, _C4, _C5 = -0.5, 4.1666667e-2, -1.3888889e-3, 2.4801587e-5, -2.7557319e-7


def _sincos(x):
    y = x * _INV_PI + 0.5
    ki = y.astype(jnp.int32)
    kf = ki.astype(jnp.float32)
    adj = kf > y
    kf = kf - jnp.where(adj, 1.0, 0.0)
    ki = ki - jnp.where(adj, 1, 0)
    r = x - kf * _PI_HI
    r = r - kf * _PI_LO
    r2 = r * r
    ps = (((_S4 * r2 + _S3) * r2 + _S2) * r2 + _S1) * r2 + 1.0
    pc = ((((_C5 * r2 + _C4) * r2 + _C3) * r2 + _C2) * r2 + _C1) * r2 + 1.0
    sinr = r * ps
    sgn = jnp.where((ki & 1) != 0, -1.0, 1.0)
    return sinr, pc, sgn


def _body(table_hbm, t_hbm, gx_hbm, gy_hbm, gz_hbm, re_hbm, im_hbm,
          t_A, idx_A, frac_A, rows_A, t_B, idx_B, frac_B, rows_B,
          gx_v, gy_v, gz_v, re_v, im_v, sem_A, sem_B):
    wid = lax.axis_index("s") * NC + lax.axis_index("c")
    w_base = wid * PER_W
    bufs = {0: (t_A, idx_A, frac_A, rows_A, sem_A),
            1: (t_B, idx_B, frac_B, rows_B, sem_B)}

    def prep_fire(p, base):
        t_v, idx_v, frac_v, rows_v, sem = bufs[p]
        pltpu.sync_copy(t_hbm.at[pl.ds(base, CHUNK)], t_v)

        @plsc.parallel_loop(0, CHUNK, L, unroll=4)
        def pass1(i):
            tv = t_v[pl.ds(i, L)]
            ti = tv.astype(jnp.int32)
            ti = jnp.minimum(jnp.maximum(ti, 0), N - 2)
            idx_v[pl.ds(i, L)] = ti
            frac_v[pl.ds(i, L)] = tv - ti.astype(jnp.float32)

        for d in range(N_DMA):
            pltpu.async_copy(
                table_hbm.at[idx_v.at[pl.ds(d * DMA_SLICE, DMA_SLICE)]],
                rows_v.at[pl.ds(d * DMA_SLICE, DMA_SLICE)],
                sem)

    def finish(p, base):
        t_v, idx_v, frac_v, rows_v, sem = bufs[p]
        for d in range(N_DMA):
            pltpu.make_async_copy(
                table_hbm.at[idx_v.at[pl.ds(d * DMA_SLICE, DMA_SLICE)]],
                rows_v.at[pl.ds(d * DMA_SLICE, DMA_SLICE)],
                sem).wait()

        @plsc.parallel_loop(0, CHUNK, L, unroll=4)
        def pass2(i):
            sl = pl.ds(i, L)
            fr = frac_v[sl]
            ri = lax.iota(jnp.int32, L) + i

            def col(cc):
                ci = jnp.full((L,), cc, jnp.int32)
                return plsc.load_gather(rows_v, [ri, ci])

            def lerp(cc):
                a = col(cc)
                b = col(cc + 5)
                return a + fr * (b - a)

            gx_v[sl] = lerp(0)
            gy_v[sl] = lerp(1)
            gz_v[sl] = lerp(2)
            amp = lerp(3)
            ph = lerp(4)
            sinr, cosr, sgn = _sincos(ph)
            amps = amp * sgn
            re_v[sl] = amps * cosr
            im_v[sl] = amps * sinr

        pltpu.sync_copy(gx_v, gx_hbm.at[pl.ds(base, CHUNK)])
        pltpu.sync_copy(gy_v, gy_hbm.at[pl.ds(base, CHUNK)])
        pltpu.sync_copy(gz_v, gz_hbm.at[pl.ds(base, CHUNK)])
        pltpu.sync_copy(re_v, re_hbm.at[pl.ds(base, CHUNK)])
        pltpu.sync_copy(im_v, im_hbm.at[pl.ds(base, CHUNK)])

    prep_fire(0, w_base)

    def pair_body(k, carry):
        base_a = w_base + 2 * k * CHUNK
        prep_fire(1, base_a + CHUNK)
        finish(0, base_a)

        @pl.when(k < N_CHUNKS // 2 - 1)
        def _():
            prep_fire(0, base_a + 2 * CHUNK)

        finish(1, base_a + CHUNK)
        return carry

    lax.fori_loop(0, N_CHUNKS // 2, pair_body, 0)


_mesh = plsc.VectorSubcoreMesh(
    core_axis_name="c", subcore_axis_name="s", num_cores=NC, num_subcores=NS)

_sc_interp = functools.partial(
    pl.kernel,
    out_type=[jax.ShapeDtypeStruct((T,), jnp.float32)] * 5,
    mesh=_mesh,
    compiler_params=pltpu.CompilerParams(
        use_tc_tiling_on_sc=False, needs_layout_passes=False),
    scratch_types=[
        pltpu.VMEM((CHUNK,), jnp.float32),
        pltpu.VMEM((CHUNK,), jnp.int32),
        pltpu.VMEM((CHUNK,), jnp.float32),
        pltpu.VMEM((CHUNK, 16), jnp.float32),
        pltpu.VMEM((CHUNK,), jnp.float32),
        pltpu.VMEM((CHUNK,), jnp.int32),
        pltpu.VMEM((CHUNK,), jnp.float32),
        pltpu.VMEM((CHUNK, 16), jnp.float32),
        pltpu.VMEM((CHUNK,), jnp.float32),
        pltpu.VMEM((CHUNK,), jnp.float32),
        pltpu.VMEM((CHUNK,), jnp.float32),
        pltpu.VMEM((CHUNK,), jnp.float32),
        pltpu.VMEM((CHUNK,), jnp.float32),
        pltpu.SemaphoreType.DMA,
        pltpu.SemaphoreType.DMA,
    ],
)(_body)


_BT = 65536


def _stack_body(gx_ref, gy_ref, gz_ref, out_ref):
    out_ref[0, :] = gx_ref[:]
    out_ref[1, :] = gy_ref[:]
    out_ref[2, :] = gz_ref[:]


_stack3 = pl.pallas_call(
    _stack_body,
    grid=(T // _BT,),
    in_specs=[pl.BlockSpec((_BT,), lambda j: (j,))] * 3,
    out_specs=pl.BlockSpec((3, _BT), lambda j: (0, j)),
    out_shape=jax.ShapeDtypeStruct((3, T), jnp.float32),
)


def kernel(time_points, gradients, rf_amplitude, rf_phase, adc_mask, t):
    rows5 = jnp.concatenate(
        [gradients, rf_amplitude[:, None], rf_phase[:, None]], axis=1)
    nxt = jnp.concatenate([rows5[1:], rows5[-1:]], axis=0)
    packed = jnp.concatenate(
        [rows5, nxt, jnp.zeros((N, 6), jnp.float32)], axis=1)
    gx, gy, gz, re, im = _sc_interp(packed, t)
    g = _stack3(gx, gy, gz)
    rf = lax.complex(re, im)
    return (g, rf)


# speedup vs baseline: 1.2831x; 1.0063x over previous
import functools

import jax
import jax.numpy as jnp
from jax import lax
from jax.experimental import pallas as pl
from jax.experimental.pallas import tpu as pltpu
from jax.experimental.pallas import tpu_sc as plsc

N = 65536
T = 2097152

NC = 2
NS = 16
L = 16
NW = NC * NS

PER_W = T // NW
CHUNK = 2048
N_CHUNKS = PER_W // CHUNK
GROUPS = CHUNK // L
DMA_SLICE = 128
N_DMA = CHUNK // DMA_SLICE

_INV_PI = 0.3183098861837907
_PI_HI = 3.140625
_PI_LO = 9.676535897932795e-4
_S1, _S2, _S3 = -1.6666667e-1, 8.3333333e-3, -1.9841270e-4
_C1, _C2, _C3, _C4 = -0.5, 4.1666667e-2, -1.3888889e-3, 2.4801587e-5
_MAGIC = 12582912.0


def _sincos(x):
    kf = (x * _INV_PI + _MAGIC) - _MAGIC
    ki = kf.astype(jnp.int32)
    r = x - kf * _PI_HI
    r = r - kf * _PI_LO
    r2 = r * r
    ps = ((_S3 * r2 + _S2) * r2 + _S1) * r2 + 1.0
    pc = (((_C4 * r2 + _C3) * r2 + _C2) * r2 + _C1) * r2 + 1.0
    sinr = r * ps
    sgn = jnp.where((ki & 1) != 0, -1.0, 1.0)
    return sinr, pc, sgn


def _body(table_hbm, t_hbm, gx_hbm, gy_hbm, gz_hbm, re_hbm, im_hbm,
          t_A, idx_A, frac_A, rows_A, t_B, idx_B, frac_B, rows_B,
          gx_v, gy_v, gz_v, re_v, im_v, sem_A, sem_B):
    wid = lax.axis_index("s") * NC + lax.axis_index("c")
    w_base = wid * PER_W
    bufs = {0: (t_A, idx_A, frac_A, rows_A, sem_A),
            1: (t_B, idx_B, frac_B, rows_B, sem_B)}

    def prep_fire(p, base):
        t_v, idx_v, frac_v, rows_v, sem = bufs[p]
        pltpu.sync_copy(t_hbm.at[pl.ds(base, CHUNK)], t_v)

        @plsc.parallel_loop(0, CHUNK, L, unroll=4)
        def pass1(i):
            tv = t_v[pl.ds(i, L)]
            ti = tv.astype(jnp.int32)
            ti = jnp.minimum(jnp.maximum(ti, 0), N - 2)
            idx_v[pl.ds(i, L)] = ti
            frac_v[pl.ds(i, L)] = tv - ti.astype(jnp.float32)

        for d in range(N_DMA):
            pltpu.async_copy(
                table_hbm.at[idx_v.at[pl.ds(d * DMA_SLICE, DMA_SLICE)]],
                rows_v.at[pl.ds(d * DMA_SLICE, DMA_SLICE)],
                sem)

    def finish(p, base):
        t_v, idx_v, frac_v, rows_v, sem = bufs[p]
        for d in range(N_DMA):
            pltpu.make_async_copy(
                table_hbm.at[idx_v.at[pl.ds(d * DMA_SLICE, DMA_SLICE)]],
                rows_v.at[pl.ds(d * DMA_SLICE, DMA_SLICE)],
                sem).wait()

        @plsc.parallel_loop(0, CHUNK, L, unroll=4)
        def pass2(i):
            sl = pl.ds(i, L)
            fr = frac_v[sl]
            ri = lax.iota(jnp.int32, L) + i

            def col(cc):
                ci = jnp.full((L,), cc, jnp.int32)
                return plsc.load_gather(rows_v, [ri, ci])

            def lerp(cc):
                a = col(cc)
                b = col(cc + 5)
                return a + fr * (b - a)

            gx_v[sl] = lerp(0)
            gy_v[sl] = lerp(1)
            gz_v[sl] = lerp(2)
            amp = lerp(3)
            ph = lerp(4)
            sinr, cosr, sgn = _sincos(ph)
            amps = amp * sgn
            re_v[sl] = amps * cosr
            im_v[sl] = amps * sinr

        pltpu.sync_copy(gx_v, gx_hbm.at[pl.ds(base, CHUNK)])
        pltpu.sync_copy(gy_v, gy_hbm.at[pl.ds(base, CHUNK)])
        pltpu.sync_copy(gz_v, gz_hbm.at[pl.ds(base, CHUNK)])
        pltpu.sync_copy(re_v, re_hbm.at[pl.ds(base, CHUNK)])
        pltpu.sync_copy(im_v, im_hbm.at[pl.ds(base, CHUNK)])

    prep_fire(0, w_base)

    def pair_body(k, carry):
        base_a = w_base + 2 * k * CHUNK
        prep_fire(1, base_a + CHUNK)
        finish(0, base_a)

        @pl.when(k < N_CHUNKS // 2 - 1)
        def _():
            prep_fire(0, base_a + 2 * CHUNK)

        finish(1, base_a + CHUNK)
        return carry

    lax.fori_loop(0, N_CHUNKS // 2, pair_body, 0)


_mesh = plsc.VectorSubcoreMesh(
    core_axis_name="c", subcore_axis_name="s", num_cores=NC, num_subcores=NS)

_sc_interp = functools.partial(
    pl.kernel,
    out_type=[jax.ShapeDtypeStruct((T,), jnp.float32)] * 5,
    mesh=_mesh,
    compiler_params=pltpu.CompilerParams(
        use_tc_tiling_on_sc=False, needs_layout_passes=False),
    scratch_types=[
        pltpu.VMEM((CHUNK,), jnp.float32),
        pltpu.VMEM((CHUNK,), jnp.int32),
        pltpu.VMEM((CHUNK,), jnp.float32),
        pltpu.VMEM((CHUNK, 16), jnp.float32),
        pltpu.VMEM((CHUNK,), jnp.float32),
        pltpu.VMEM((CHUNK,), jnp.int32),
        pltpu.VMEM((CHUNK,), jnp.float32),
        pltpu.VMEM((CHUNK, 16), jnp.float32),
        pltpu.VMEM((CHUNK,), jnp.float32),
        pltpu.VMEM((CHUNK,), jnp.float32),
        pltpu.VMEM((CHUNK,), jnp.float32),
        pltpu.VMEM((CHUNK,), jnp.float32),
        pltpu.VMEM((CHUNK,), jnp.float32),
        pltpu.SemaphoreType.DMA,
        pltpu.SemaphoreType.DMA,
    ],
)(_body)


_BT = 65536


def _stack_body(gx_ref, gy_ref, gz_ref, out_ref):
    out_ref[0, :] = gx_ref[:]
    out_ref[1, :] = gy_ref[:]
    out_ref[2, :] = gz_ref[:]


_stack3 = pl.pallas_call(
    _stack_body,
    grid=(T // _BT,),
    in_specs=[pl.BlockSpec((_BT,), lambda j: (j,))] * 3,
    out_specs=pl.BlockSpec((3, _BT), lambda j: (0, j)),
    out_shape=jax.ShapeDtypeStruct((3, T), jnp.float32),
)


def kernel(time_points, gradients, rf_amplitude, rf_phase, adc_mask, t):
    rows5 = jnp.concatenate(
        [gradients, rf_amplitude[:, None], rf_phase[:, None]], axis=1)
    nxt = jnp.concatenate([rows5[1:], rows5[-1:]], axis=0)
    packed = jnp.concatenate(
        [rows5, nxt, jnp.zeros((N, 6), jnp.float32)], axis=1)
    gx, gy, gz, re, im = _sc_interp(packed, t)
    g = _stack3(gx, gy, gz)
    rf = lax.complex(re, im)
    return (g, rf)
